# baseline (speedup 1.0000x reference)
import functools

import jax
import jax.numpy as jnp
from jax import lax
from jax.experimental import pallas as pl
from jax.experimental.pallas import tpu as pltpu
from jax.experimental.pallas import tpu_sc as plsc

NN = 10000
NE = 320000
D = 128
NC = 2
NS = 16
NW = NC * NS
EPW = NE // NW
CB = 80
NCHUNK = EPW // CB
WR = 40
F32 = jnp.float32

_MESH = plsc.VectorSubcoreMesh(core_axis_name="c", subcore_axis_name="s")
_SC_PARAMS = pltpu.CompilerParams(needs_layout_passes=False)


def _sds(shape, dtype=F32):
    return jax.ShapeDtypeStruct(shape, dtype)


NG = CB // 16


def _prep_body(xx_hbm, xy_hbm, xz_hbm, row_hbm, col_hbm, ea_hbm, cd_hbm,
               xxv, xyv, xzv, ridx0, cidx0, eab0, cdb0,
               ridx1, cidx1, eab1, cdb1, semL0, semL1, semW0, semW1):
    cid = lax.axis_index("c")
    sid = lax.axis_index("s")
    base = (cid * NS + sid) * EPW
    ridx, cidx = (ridx0, ridx1), (cidx0, cidx1)
    eab, cdb = (eab0, eab1), (cdb0, cdb1)
    semL, semW = (semL0, semL1), (semW0, semW1)
    pltpu.sync_copy(xx_hbm, xxv)
    pltpu.sync_copy(xy_hbm, xyv)
    pltpu.sync_copy(xz_hbm, xzv)

    for bb in (0, 1):
        def z(j, c, bb=bb):
            cdb[bb][j] = jnp.zeros((16,), F32)
            return c
        lax.fori_loop(0, CB, z, 0)
    lane16 = lax.iota(jnp.int32, 16)

    def issue(k, b):
        off = base + k * CB
        pltpu.async_copy(row_hbm.at[pl.ds(off, CB)], ridx[b], semL[b])
        pltpu.async_copy(col_hbm.at[pl.ds(off, CB)], cidx[b], semL[b])
        pltpu.async_copy(ea_hbm.at[pl.ds(off, CB)], eab[b], semL[b])

    issue(0, 0)

    def outer(t, carry):
        for b in (0, 1):
            k = 2 * t + b
            nb = 1 - b

            @pl.when(k + 1 < NCHUNK)
            def _issue_next():
                issue(k + 1, nb)

            @pl.when(k < NCHUNK)
            def _drain():
                pltpu.make_async_copy(
                    row_hbm.at[pl.ds(0, CB)], ridx[b], semL[b]).wait()
                pltpu.make_async_copy(
                    row_hbm.at[pl.ds(0, CB)], cidx[b], semL[b]).wait()
                pltpu.make_async_copy(
                    ea_hbm.at[pl.ds(0, CB)], eab[b], semL[b]).wait()

                @pl.when(k >= 2)
                def _wait_write():
                    pltpu.make_async_copy(
                        cdb[b], cd_hbm.at[0, pl.ds(0, CB)], semW[b]).wait()

                def grp(j, c2):
                    r = ridx[b][pl.ds(j * 16, 16)]
                    cc = cidx[b][pl.ds(j * 16, 16)]
                    erow = j * 16 + lane16
                    for comp, ref in enumerate((xxv, xyv, xzv)):
                        d = (plsc.load_gather(ref, [r])
                             - plsc.load_gather(ref, [cc]))
                        plsc.store_scatter(
                            cdb[b], [erow, jnp.full((16,), comp, jnp.int32)],
                            d)
                    for comp in range(4):
                        v = plsc.load_gather(
                            eab[b], [erow, jnp.full((16,), comp, jnp.int32)])
                        plsc.store_scatter(
                            cdb[b],
                            [erow, jnp.full((16,), comp + 3, jnp.int32)], v)
                    return c2

                lax.fori_loop(0, NG, grp, 0)
                off_l = sid * EPW + k * CB
                pltpu.async_copy(
                    cdb[b], cd_hbm.at[cid, pl.ds(off_l, CB)], semW[b])
        return carry

    lax.fori_loop(0, (NCHUNK + 1) // 2, outer, 0)
    pltpu.make_async_copy(cdb[0], cd_hbm.at[0, pl.ds(0, CB)], semW[0]).wait()
    pltpu.make_async_copy(cdb[1], cd_hbm.at[0, pl.ds(0, CB)], semW[1]).wait()


@jax.jit
def _prep(xx, xy, xz, row, col, ea):
    return pl.kernel(
        _prep_body,
        out_type=_sds((NC, NE2, 16)),
        mesh=_MESH,
        compiler_params=_SC_PARAMS,
        scratch_types=[
            pltpu.VMEM((NN,), F32),
            pltpu.VMEM((NN,), F32),
            pltpu.VMEM((NN,), F32),
            pltpu.VMEM((CB,), jnp.int32),
            pltpu.VMEM((CB,), jnp.int32),
            pltpu.VMEM((CB, 4), F32),
            pltpu.VMEM((CB, 16), F32),
            pltpu.VMEM((CB,), jnp.int32),
            pltpu.VMEM((CB,), jnp.int32),
            pltpu.VMEM((CB, 4), F32),
            pltpu.VMEM((CB, 16), F32),
            pltpu.SemaphoreType.DMA,
            pltpu.SemaphoreType.DMA,
            pltpu.SemaphoreType.DMA,
            pltpu.SemaphoreType.DMA,
        ],
    )(xx, xy, xz, row, col, ea)


def _make_gather(ne, cb):
    epw = ne // NW
    nchunk = epw // cb

    def body(ha_hbm, hb_hbm, row_hbm, col_hbm, g_hbm,
             ridx0, cidx0, ridx1, cidx1, gA0, gB0, gA1, gB1,
             semG0, semG1, semW0, semW1):
        wid = lax.axis_index("c") * NS + lax.axis_index("s")
        base = wid * epw
        ridx, cidx = (ridx0, ridx1), (cidx0, cidx1)
        gA, gB = (gA0, gA1), (gB0, gB1)
        semG, semW = (semG0, semG1), (semW0, semW1)

        def issue(k, b):
            off = base + k * cb
            pltpu.sync_copy(row_hbm.at[pl.ds(off, cb)], ridx[b])
            pltpu.sync_copy(col_hbm.at[pl.ds(off, cb)], cidx[b])
            pltpu.async_copy(ha_hbm.at[ridx[b]], gA[b], semG[b])
            pltpu.async_copy(hb_hbm.at[cidx[b]], gB[b], semG[b])

        issue(0, 0)

        def outer(t, carry):
            for b in (0, 1):
                k = 2 * t + b
                nb = 1 - b

                @pl.when(k + 1 < nchunk)
                def _issue_next():
                    @pl.when(k >= 1)
                    def _wait_write():
                        pltpu.make_async_copy(
                            gA[nb], g_hbm.at[pl.ds(0, cb)], semW[nb]).wait()
                    issue(k + 1, nb)

                @pl.when(k < nchunk)
                def _drain():
                    pltpu.make_async_copy(
                        ha_hbm.at[pl.ds(0, cb)], gA[b], semG[b]).wait()
                    pltpu.make_async_copy(
                        ha_hbm.at[pl.ds(0, cb)], gB[b], semG[b]).wait()

                    def add(j, c):
                        def addl(l, c2):
                            sl = pl.ds(l * 16, 16)
                            gA[b][j, sl] = gA[b][j, sl] + gB[b][j, sl]
                            return c2
                        return lax.fori_loop(0, D // 16, addl, c)

                    lax.fori_loop(0, cb, add, 0)
                    off = base + k * cb
                    pltpu.async_copy(gA[b], g_hbm.at[pl.ds(off, cb)], semW[b])
            return carry

        lax.fori_loop(0, (nchunk + 1) // 2, outer, 0)
        pltpu.make_async_copy(gA[0], g_hbm.at[pl.ds(0, cb)], semW[0]).wait()
        pltpu.make_async_copy(gA[1], g_hbm.at[pl.ds(0, cb)], semW[1]).wait()

    @jax.jit
    def run(ha, hb, row, col):
        return pl.kernel(
            body,
            out_type=_sds((ne, D)),
            mesh=_MESH,
            compiler_params=_SC_PARAMS,
            scratch_types=[
                pltpu.VMEM((cb,), jnp.int32),
                pltpu.VMEM((cb,), jnp.int32),
                pltpu.VMEM((cb,), jnp.int32),
                pltpu.VMEM((cb,), jnp.int32),
                pltpu.VMEM((cb, D), F32),
                pltpu.VMEM((cb, D), F32),
                pltpu.VMEM((cb, D), F32),
                pltpu.VMEM((cb, D), F32),
                pltpu.SemaphoreType.DMA,
                pltpu.SemaphoreType.DMA,
                pltpu.SemaphoreType.DMA,
                pltpu.SemaphoreType.DMA,
            ],
        )(ha, hb, row, col)

    return run


NE2 = NE // 2
CB2 = 40
_gather_h = _make_gather(NE2, CB2)


NCHN = NN // WR
NTRIP = (NCHN + NS - 1) // NS


def _zero_shared(shared, zb, sid):
    def zrow(i, c):
        def zlane(l, c2):
            zb[i, pl.ds(l * 16, 16)] = jnp.zeros((16,), F32)
            return c2
        return lax.fori_loop(0, D // 16, zlane, c)

    lax.fori_loop(0, WR, zrow, 0)

    def zcopy(t, c):
        cidx = t * NS + sid

        @pl.when(cidx < NCHN)
        def _():
            pltpu.sync_copy(zb, shared.at[pl.ds(cidx * WR, WR)])
        return c

    lax.fori_loop(0, NTRIP, zcopy, 0)


def _writeout(shared, ob, out_hbm, cid, sid):
    def wout(t, c):
        cidx = t * NS + sid

        @pl.when(cidx < NCHN)
        def _():
            r0 = cidx * WR
            pltpu.sync_copy(shared.at[pl.ds(r0, WR)], ob)
            pltpu.sync_copy(ob, out_hbm.at[cid, pl.ds(r0, WR)])
        return c

    lax.fori_loop(0, NTRIP, wout, 0)


def _make_scatter(ne, cb):
    epw = ne // NW
    nchunk = epw // cb

    def body(m2_hbm, row_hbm, agg_hbm, ridx0, mb0, ridx1, mb1, ob,
             shared, semL0, semL1, semS0, semS1):
        cid = lax.axis_index("c")
        sid = lax.axis_index("s")
        _zero_shared(shared, ob, sid)
        plsc.subcore_barrier()
        base = (cid * NS + sid) * epw
        ridx, mb = (ridx0, ridx1), (mb0, mb1)
        semL, semS = (semL0, semL1), (semS0, semS1)

        def issue(k, b):
            off = base + k * cb
            pltpu.async_copy(row_hbm.at[pl.ds(off, cb)], ridx[b], semL[b])
            pltpu.async_copy(m2_hbm.at[pl.ds(off, cb)], mb[b], semL[b])

        issue(0, 0)

        def outer(t, carry):
            for b in (0, 1):
                k = 2 * t + b
                nb = 1 - b

                @pl.when(k + 1 < nchunk)
                def _issue_next():
                    @pl.when(k >= 1)
                    def _wait_scat():
                        pltpu.make_async_copy(
                            mb[nb], shared.at[pl.ds(0, cb)], semS[nb]).wait()
                    issue(k + 1, nb)

                @pl.when(k < nchunk)
                def _drain():
                    pltpu.make_async_copy(
                        row_hbm.at[pl.ds(0, cb)], ridx[b], semL[b]).wait()
                    pltpu.make_async_copy(
                        m2_hbm.at[pl.ds(0, cb)], mb[b], semL[b]).wait()
                    pltpu.async_copy(
                        mb[b], shared.at[ridx[b]], semS[b], add=True)
            return carry

        lax.fori_loop(0, (nchunk + 1) // 2, outer, 0)
        pltpu.make_async_copy(mb[0], shared.at[pl.ds(0, cb)], semS[0]).wait()
        pltpu.make_async_copy(mb[1], shared.at[pl.ds(0, cb)], semS[1]).wait()
        plsc.subcore_barrier()
        _writeout(shared, ob, agg_hbm, cid, sid)

    @jax.jit
    def run(m2, row):
        return pl.kernel(
            body,
            out_type=_sds((NC, NN, D)),
            mesh=_MESH,
            compiler_params=_SC_PARAMS,
            scratch_types=[
                pltpu.VMEM((cb,), jnp.int32),
                pltpu.VMEM((cb, D), F32),
                pltpu.VMEM((cb,), jnp.int32),
                pltpu.VMEM((cb, D), F32),
                pltpu.VMEM((WR, D), F32),
                pltpu.VMEM_SHARED((NN, D), F32),
                pltpu.SemaphoreType.DMA,
                pltpu.SemaphoreType.DMA,
                pltpu.SemaphoreType.DMA,
                pltpu.SemaphoreType.DMA,
            ],
        )(m2, row)

    return run


_scatter_h = _make_scatter(NE2, CB2)


def _scatter_tr_body(tr0_hbm, tr1_hbm, row_hbm, xacc_hbm,
                     ridx0, trb0, ridx1, trb1, px, py, pz, pc,
                     semL0, semL1):
    cid = lax.axis_index("c")
    sid = lax.axis_index("s")
    wid = cid * NS + sid
    pacc = (px, py, pz, pc)
    for p in pacc:
        def zp(j, c, p=p):
            p[pl.ds(j * 16, 16)] = jnp.zeros((16,), F32)
            return c
        lax.fori_loop(0, NN // 16, zp, 0)
    base = sid * EPW
    ridx, trb = (ridx0, ridx1), (trb0, trb1)
    semL = (semL0, semL1)
    lane16 = lax.iota(jnp.int32, 16)

    def issue(k, b):
        off = base + k * CB
        pltpu.async_copy(
            row_hbm.at[pl.ds(cid * NE2 + off, CB)], ridx[b], semL[b])

        @pl.when(cid == 0)
        def _h0():
            pltpu.async_copy(tr0_hbm.at[pl.ds(off, CB)], trb[b], semL[b])

        @pl.when(cid == 1)
        def _h1():
            pltpu.async_copy(tr1_hbm.at[pl.ds(off, CB)], trb[b], semL[b])

    issue(0, 0)

    def outer(t, carry):
        for b in (0, 1):
            k = 2 * t + b
            nb = 1 - b

            @pl.when(k + 1 < NCHUNK)
            def _issue_next():
                issue(k + 1, nb)

            @pl.when(k < NCHUNK)
            def _drain():
                pltpu.make_async_copy(
                    row_hbm.at[pl.ds(0, CB)], ridx[b], semL[b]).wait()
                pltpu.make_async_copy(
                    tr0_hbm.at[pl.ds(0, CB)], trb[b], semL[b]).wait()

                def grp(j, c2):
                    idxv = ridx[b][pl.ds(j * 16, 16)]
                    erow = j * 16 + lane16
                    for comp, p in enumerate(pacc):
                        v = plsc.load_gather(
                            trb[b], [erow, jnp.full((16,), comp, jnp.int32)])
                        plsc.addupdate_scatter(p, [idxv], v)
                    return c2

                lax.fori_loop(0, NG, grp, 0)
        return carry

    lax.fori_loop(0, (NCHUNK + 1) // 2, outer, 0)
    for comp, p in enumerate(pacc):
        pltpu.sync_copy(p, xacc_hbm.at[comp, wid])


@jax.jit
def _scatter_tr(tr0, tr1, row):
    return pl.kernel(
        _scatter_tr_body,
        out_type=_sds((4, NW, NN)),
        mesh=_MESH,
        compiler_params=_SC_PARAMS,
        scratch_types=[
            pltpu.VMEM((CB,), jnp.int32),
            pltpu.VMEM((CB, 16), F32),
            pltpu.VMEM((CB,), jnp.int32),
            pltpu.VMEM((CB, 16), F32),
            pltpu.VMEM((NN,), F32),
            pltpu.VMEM((NN,), F32),
            pltpu.VMEM((NN,), F32),
            pltpu.VMEM((NN,), F32),
            pltpu.SemaphoreType.DMA,
            pltpu.SemaphoreType.DMA,
        ],
    )(tr0, tr1, row)


BN = 1000
BE = 4000


def _full(shape):
    return pl.BlockSpec(shape, lambda i: (0,) * len(shape))


def _pre_body(h_ref, w1at, w1bt, b1, ha_ref, hb_ref):
    h = h_ref[...]
    ha_ref[...] = jnp.dot(h, w1at[...], preferred_element_type=F32) + b1[...]
    hb_ref[...] = jnp.dot(h, w1bt[...], preferred_element_type=F32)


@jax.jit
def _pre(h, w1at, w1bt, b1):
    return pl.pallas_call(
        _pre_body,
        grid=(NN // BN,),
        in_specs=[
            pl.BlockSpec((BN, D), lambda i: (i, 0)),
            _full((D, D)), _full((D, D)), _full((1, D)),
        ],
        out_specs=[pl.BlockSpec((BN, D), lambda i: (i, 0))] * 2,
        out_shape=[_sds((NN, D))] * 2,
    )(h, w1at, w1bt, b1)


def _silu(v):
    return v * (0.5 * jnp.tanh(0.5 * v) + 0.5)


def _radial_of(cd, shape):
    lane = lax.broadcasted_iota(jnp.int32, shape, 1)
    return jnp.sum(jnp.where(lane < 3, cd * cd, 0.0), axis=1, keepdims=True)


def _edge_core(g_ref, cd_ref, wrt, w1et, w2t, b2):
    g = g_ref[...]
    cd = cd_ref[...]
    radial = _radial_of(cd, (BE, 16))
    pre = g + radial * wrt[...] + jnp.dot(
        cd, w1et[...], preferred_element_type=F32)
    m = _silu(pre.astype(jnp.bfloat16))
    return _silu(jnp.dot(m, w2t[...], preferred_element_type=F32)
                 + b2[...])


def _edge_body(g_ref, cd_ref, wrt, w1et, w2t, b2, m2_ref):
    m2_ref[...] = _edge_core(g_ref, cd_ref, wrt, w1et, w2t, b2)


@jax.jit
def _edge(g, cd, wrt, w1et, w2t, b2):
    return pl.pallas_call(
        _edge_body,
        grid=(NE2 // BE,),
        in_specs=[
            pl.BlockSpec((BE, D), lambda i: (i, 0)),
            pl.BlockSpec((BE, 16), lambda i: (i, 0)),
            _full((1, D)), _full((16, D)), _full((D, D)), _full((1, D)),
        ],
        out_specs=pl.BlockSpec((BE, D), lambda i: (i, 0)),
        out_shape=_sds((NE2, D)),
    )(g, cd, wrt, w1et, w2t, b2)


def _edge_last_body(g_ref, cd_ref, wrt, w1et, w2t, b2,
                    cw1t, cb1, cw2, m2_ref, tr_ref):
    m2 = _edge_core(g_ref, cd_ref, wrt, w1et, w2t, b2)
    m2_ref[...] = m2
    t = _silu(jnp.dot(m2, cw1t[...], preferred_element_type=F32)
              + cb1[...])
    phi = jnp.sum(t * cw2[...], axis=1, keepdims=True)
    lane = lax.broadcasted_iota(jnp.int32, (BE, 16), 1)
    tr_ref[...] = jnp.where(lane < 3, cd_ref[...] * phi,
                            jnp.where(lane == 3, 1.0, 0.0))


@jax.jit
def _edge_last(g, cd, wrt, w1et, w2t, b2, cw1t, cb1, cw2):
    return pl.pallas_call(
        _edge_last_body,
        grid=(NE2 // BE,),
        in_specs=[
            pl.BlockSpec((BE, D), lambda i: (i, 0)),
            pl.BlockSpec((BE, 16), lambda i: (i, 0)),
            _full((1, D)), _full((16, D)), _full((D, D)), _full((1, D)),
            _full((D, D)), _full((1, D)), _full((1, D)),
        ],
        out_specs=[pl.BlockSpec((BE, D), lambda i: (i, 0)),
                   pl.BlockSpec((BE, 16), lambda i: (i, 0))],
        out_shape=[_sds((NE2, D)), _sds((NE2, 16))],
    )(g, cd, wrt, w1et, w2t, b2, cw1t, cb1, cw2)


def _node_core(h_ref, a0_ref, a1_ref, a2_ref, a3_ref,
               w1ht, w1at, nb1, w2t, nb2):
    h = h_ref[...]
    agg = a0_ref[0] + a1_ref[0] + a2_ref[0] + a3_ref[0]
    o1 = _silu(jnp.dot(h, w1ht[...], preferred_element_type=F32)
                     + jnp.dot(agg, w1at[...], preferred_element_type=F32)
                     + nb1[...])
    return h + jnp.dot(o1, w2t[...], preferred_element_type=F32) + nb2[...]


def _node_body(h_ref, a0_ref, a1_ref, a2_ref, a3_ref,
               w1ht, w1at, nb1, w2t, nb2,
               ew1at, ew1bt, eb1, hn_ref, ha_ref, hb_ref):
    hn = _node_core(h_ref, a0_ref, a1_ref, a2_ref, a3_ref,
                    w1ht, w1at, nb1, w2t, nb2)
    hn_ref[...] = hn
    ha_ref[...] = jnp.dot(hn, ew1at[...], preferred_element_type=F32) + eb1[...]
    hb_ref[...] = jnp.dot(hn, ew1bt[...], preferred_element_type=F32)


_AGG_SPECS = [
    pl.BlockSpec((1, BN, D), lambda i: (0, i, 0)),
    pl.BlockSpec((1, BN, D), lambda i: (1, i, 0)),
    pl.BlockSpec((1, BN, D), lambda i: (0, i, 0)),
    pl.BlockSpec((1, BN, D), lambda i: (1, i, 0)),
]


@jax.jit
def _node(h, aggp0, aggp1, w1ht, w1at, nb1, w2t, nb2, ew1at, ew1bt, eb1):
    return pl.pallas_call(
        _node_body,
        grid=(NN // BN,),
        in_specs=[
            pl.BlockSpec((BN, D), lambda i: (i, 0)),
            *_AGG_SPECS,
            _full((D, D)), _full((D, D)), _full((1, D)),
            _full((D, D)), _full((1, D)),
            _full((D, D)), _full((D, D)), _full((1, D)),
        ],
        out_specs=[pl.BlockSpec((BN, D), lambda i: (i, 0))] * 3,
        out_shape=[_sds((NN, D))] * 3,
    )(h, aggp0, aggp0, aggp1, aggp1,
      w1ht, w1at, nb1, w2t, nb2, ew1at, ew1bt, eb1)


def _node_last_body(h_ref, a0_ref, a1_ref, a2_ref, a3_ref,
                    w1ht, w1at, nb1, w2t, nb2,
                    xt_ref, xap_ref, hn_ref, xn_ref):
    hn_ref[...] = _node_core(h_ref, a0_ref, a1_ref, a2_ref, a3_ref,
                             w1ht, w1at, nb1, w2t, nb2)
    xab = xap_ref[...]
    parts = [jnp.sum(xab[c * NW:(c + 1) * NW], axis=0, keepdims=True)
             for c in range(4)]
    cnt = jnp.clip(parts[3], 1.0, None)
    sxyz = jnp.concatenate(
        parts[:3] + [jnp.zeros((5, NN), F32)], axis=0)
    row = lax.broadcasted_iota(jnp.int32, (8, NN), 0)
    xn_ref[...] = xt_ref[...] + jnp.where(row < 3, sxyz / cnt, 0.0)


@jax.jit
def _node_last(h, aggp0, aggp1, w1ht, w1at, nb1, w2t, nb2, xt, xaccp):
    return pl.pallas_call(
        _node_last_body,
        grid=(NN // BN,),
        in_specs=[
            pl.BlockSpec((BN, D), lambda i: (i, 0)),
            *_AGG_SPECS,
            _full((D, D)), _full((D, D)), _full((1, D)),
            _full((D, D)), _full((1, D)),
            _full((8, NN)),
            _full((4 * NW, NN)),
        ],
        out_specs=[pl.BlockSpec((BN, D), lambda i: (i, 0)),
                   _full((8, NN))],
        out_shape=[_sds((NN, D)), _sds((8, NN))],
    )(h, aggp0, aggp0, aggp1, aggp1,
      w1ht, w1at, nb1, w2t, nb2, xt, xaccp)


def kernel(h, x, edges, edge_attr, params):
    row, col = edges[0], edges[1]
    xt = jnp.pad(x.T, ((0, 5), (0, 0)))
    cdp = _prep(x[:, 0], x[:, 1], x[:, 2], row, col, edge_attr)

    def esplit(p):
        w1 = p['e_w1']
        w1e16 = jnp.zeros((16, D), F32).at[3:7].set(w1[:, 2 * D + 1:].T)
        return (w1[:, :D].T, w1[:, D:2 * D].T, w1[:, 2 * D:2 * D + 1].T,
                w1e16, p['e_b1'].reshape(1, D))

    w1at, w1bt, wrt, w1et, eb1 = esplit(params[0])
    ha, hb = _pre(h, w1at, w1bt, eb1)

    rows = (row[:NE2], row[NE2:])
    cols = (col[:NE2], col[NE2:])
    cds = (cdp[0], cdp[1])

    hn = h
    for i, p in enumerate(params):
        w2t = p['e_w2'].T.astype(jnp.bfloat16)
        eb2 = p['e_b2'].reshape(1, D)
        nw1 = p['n_w1']
        nw1ht, nw1at = nw1[:, :D].T, nw1[:, D:].T
        nb1 = p['n_b1'].reshape(1, D)
        nw2t = p['n_w2'].T
        nb2 = p['n_b2'].reshape(1, D)
        gs = [_gather_h(ha, hb, rows[hh], cols[hh]) for hh in range(2)]
        if i < len(params) - 1:
            aggps = []
            for hh in range(2):
                m2 = _edge(gs[hh], cds[hh], wrt, w1et, w2t, eb2)
                aggps.append(_scatter_h(m2, rows[hh]))
            w1at, w1bt, wrt, w1et, eb1 = esplit(params[i + 1])
            hn, ha, hb = _node(hn, aggps[0], aggps[1], nw1ht, nw1at, nb1,
                               nw2t, nb2, w1at, w1bt, eb1)
        else:
            cw1t = p['c_w1'].T
            cb1 = p['c_b1'].reshape(1, D)
            cw2 = p['c_w2'].reshape(1, D)
            aggps, trs = [], []
            for hh in range(2):
                m2, tr = _edge_last(gs[hh], cds[hh], wrt, w1et, w2t, eb2,
                                    cw1t, cb1, cw2)
                trs.append(tr)
                aggps.append(_scatter_h(m2, rows[hh]))
            xaccp = _scatter_tr(trs[0], trs[1], row)
            hn, xnt = _node_last(hn, aggps[0], aggps[1], nw1ht, nw1at, nb1,
                                 nw2t, nb2, xt, xaccp.reshape(4 * NW, NN))
    return hn, xnt[:3].T


# speedup vs baseline: 1.0165x; 1.0165x over previous
import functools

import jax
import jax.numpy as jnp
from jax import lax
from jax.experimental import pallas as pl
from jax.experimental.pallas import tpu as pltpu
from jax.experimental.pallas import tpu_sc as plsc

NN = 10000
NE = 320000
D = 128
NC = 2
NS = 16
NW = NC * NS
EPW = NE // NW
CB = 80
NCHUNK = EPW // CB
WR = 40
F32 = jnp.float32

_MESH = plsc.VectorSubcoreMesh(core_axis_name="c", subcore_axis_name="s")
_SC_PARAMS = pltpu.CompilerParams(needs_layout_passes=False)


def _sds(shape, dtype=F32):
    return jax.ShapeDtypeStruct(shape, dtype)


NG = CB // 16


def _prep_body(xx_hbm, xy_hbm, xz_hbm, row_hbm, col_hbm, ea_hbm, cd_hbm,
               xxv, xyv, xzv, ridx0, cidx0, eab0, cdb0,
               ridx1, cidx1, eab1, cdb1, semL0, semL1, semW0, semW1):
    cid = lax.axis_index("c")
    sid = lax.axis_index("s")
    base = (cid * NS + sid) * EPW
    ridx, cidx = (ridx0, ridx1), (cidx0, cidx1)
    eab, cdb = (eab0, eab1), (cdb0, cdb1)
    semL, semW = (semL0, semL1), (semW0, semW1)
    pltpu.sync_copy(xx_hbm, xxv)
    pltpu.sync_copy(xy_hbm, xyv)
    pltpu.sync_copy(xz_hbm, xzv)

    for bb in (0, 1):
        def z(j, c, bb=bb):
            cdb[bb][j] = jnp.zeros((16,), F32)
            return c
        lax.fori_loop(0, CB, z, 0)
    lane16 = lax.iota(jnp.int32, 16)

    def issue(k, b):
        off = base + k * CB
        pltpu.async_copy(row_hbm.at[pl.ds(off, CB)], ridx[b], semL[b])
        pltpu.async_copy(col_hbm.at[pl.ds(off, CB)], cidx[b], semL[b])
        pltpu.async_copy(ea_hbm.at[pl.ds(off, CB)], eab[b], semL[b])

    issue(0, 0)

    def outer(t, carry):
        for b in (0, 1):
            k = 2 * t + b
            nb = 1 - b

            @pl.when(k + 1 < NCHUNK)
            def _issue_next():
                issue(k + 1, nb)

            @pl.when(k < NCHUNK)
            def _drain():
                pltpu.make_async_copy(
                    row_hbm.at[pl.ds(0, CB)], ridx[b], semL[b]).wait()
                pltpu.make_async_copy(
                    row_hbm.at[pl.ds(0, CB)], cidx[b], semL[b]).wait()
                pltpu.make_async_copy(
                    ea_hbm.at[pl.ds(0, CB)], eab[b], semL[b]).wait()

                @pl.when(k >= 2)
                def _wait_write():
                    pltpu.make_async_copy(
                        cdb[b], cd_hbm.at[0, pl.ds(0, CB)], semW[b]).wait()

                def grp(j, c2):
                    r = ridx[b][pl.ds(j * 16, 16)]
                    cc = cidx[b][pl.ds(j * 16, 16)]
                    erow = j * 16 + lane16
                    for comp, ref in enumerate((xxv, xyv, xzv)):
                        d = (plsc.load_gather(ref, [r])
                             - plsc.load_gather(ref, [cc]))
                        plsc.store_scatter(
                            cdb[b], [erow, jnp.full((16,), comp, jnp.int32)],
                            d)
                    for comp in range(4):
                        v = plsc.load_gather(
                            eab[b], [erow, jnp.full((16,), comp, jnp.int32)])
                        plsc.store_scatter(
                            cdb[b],
                            [erow, jnp.full((16,), comp + 3, jnp.int32)], v)
                    return c2

                lax.fori_loop(0, NG, grp, 0)
                off_l = sid * EPW + k * CB
                pltpu.async_copy(
                    cdb[b], cd_hbm.at[cid, pl.ds(off_l, CB)], semW[b])
        return carry

    lax.fori_loop(0, (NCHUNK + 1) // 2, outer, 0)
    pltpu.make_async_copy(cdb[0], cd_hbm.at[0, pl.ds(0, CB)], semW[0]).wait()
    pltpu.make_async_copy(cdb[1], cd_hbm.at[0, pl.ds(0, CB)], semW[1]).wait()


@jax.jit
def _prep(xx, xy, xz, row, col, ea):
    return pl.kernel(
        _prep_body,
        out_type=_sds((NC, NE2, 16)),
        mesh=_MESH,
        compiler_params=_SC_PARAMS,
        scratch_types=[
            pltpu.VMEM((NN,), F32),
            pltpu.VMEM((NN,), F32),
            pltpu.VMEM((NN,), F32),
            pltpu.VMEM((CB,), jnp.int32),
            pltpu.VMEM((CB,), jnp.int32),
            pltpu.VMEM((CB, 4), F32),
            pltpu.VMEM((CB, 16), F32),
            pltpu.VMEM((CB,), jnp.int32),
            pltpu.VMEM((CB,), jnp.int32),
            pltpu.VMEM((CB, 4), F32),
            pltpu.VMEM((CB, 16), F32),
            pltpu.SemaphoreType.DMA,
            pltpu.SemaphoreType.DMA,
            pltpu.SemaphoreType.DMA,
            pltpu.SemaphoreType.DMA,
        ],
    )(xx, xy, xz, row, col, ea)


def _make_gather(ne, cb, half_off):
    epw = ne // NW
    nchunk = epw // cb

    def body(ha_hbm, hb_hbm, row_hbm, col_hbm, g_hbm,
             ridx0, cidx0, ridx1, cidx1, gA0, gB0, gA1, gB1,
             semG0, semG1, semW0, semW1):
        wid = lax.axis_index("c") * NS + lax.axis_index("s")
        base = wid * epw
        ridx, cidx = (ridx0, ridx1), (cidx0, cidx1)
        gA, gB = (gA0, gA1), (gB0, gB1)
        semG, semW = (semG0, semG1), (semW0, semW1)

        def issue(k, b):
            off = base + k * cb
            pltpu.sync_copy(row_hbm.at[pl.ds(half_off + off, cb)], ridx[b])
            pltpu.sync_copy(col_hbm.at[pl.ds(half_off + off, cb)], cidx[b])
            pltpu.async_copy(ha_hbm.at[ridx[b]], gA[b], semG[b])
            pltpu.async_copy(hb_hbm.at[cidx[b]], gB[b], semG[b])

        issue(0, 0)

        def outer(t, carry):
            for b in (0, 1):
                k = 2 * t + b
                nb = 1 - b

                @pl.when(k + 1 < nchunk)
                def _issue_next():
                    @pl.when(k >= 1)
                    def _wait_write():
                        pltpu.make_async_copy(
                            gA[nb], g_hbm.at[pl.ds(0, cb)], semW[nb]).wait()
                    issue(k + 1, nb)

                @pl.when(k < nchunk)
                def _drain():
                    pltpu.make_async_copy(
                        ha_hbm.at[pl.ds(0, cb)], gA[b], semG[b]).wait()
                    pltpu.make_async_copy(
                        ha_hbm.at[pl.ds(0, cb)], gB[b], semG[b]).wait()

                    def add(j, c):
                        def addl(l, c2):
                            sl = pl.ds(l * 16, 16)
                            gA[b][j, sl] = gA[b][j, sl] + gB[b][j, sl]
                            return c2
                        return lax.fori_loop(0, D // 16, addl, c)

                    lax.fori_loop(0, cb, add, 0)
                    off = base + k * cb
                    pltpu.async_copy(gA[b], g_hbm.at[pl.ds(off, cb)], semW[b])
            return carry

        lax.fori_loop(0, (nchunk + 1) // 2, outer, 0)
        pltpu.make_async_copy(gA[0], g_hbm.at[pl.ds(0, cb)], semW[0]).wait()
        pltpu.make_async_copy(gA[1], g_hbm.at[pl.ds(0, cb)], semW[1]).wait()

    @jax.jit
    def run(ha, hb, row, col):
        return pl.kernel(
            body,
            out_type=_sds((ne, D)),
            mesh=_MESH,
            compiler_params=_SC_PARAMS,
            scratch_types=[
                pltpu.VMEM((cb,), jnp.int32),
                pltpu.VMEM((cb,), jnp.int32),
                pltpu.VMEM((cb,), jnp.int32),
                pltpu.VMEM((cb,), jnp.int32),
                pltpu.VMEM((cb, D), F32),
                pltpu.VMEM((cb, D), F32),
                pltpu.VMEM((cb, D), F32),
                pltpu.VMEM((cb, D), F32),
                pltpu.SemaphoreType.DMA,
                pltpu.SemaphoreType.DMA,
                pltpu.SemaphoreType.DMA,
                pltpu.SemaphoreType.DMA,
            ],
        )(ha, hb, row, col)

    return run


NE2 = NE // 2
CB2 = 40
_gather_h0 = _make_gather(NE2, CB2, 0)
_gather_h1 = _make_gather(NE2, CB2, NE2)


NCHN = NN // WR
NTRIP = (NCHN + NS - 1) // NS


def _zero_shared(shared, zb, sid):
    def zrow(i, c):
        def zlane(l, c2):
            zb[i, pl.ds(l * 16, 16)] = jnp.zeros((16,), F32)
            return c2
        return lax.fori_loop(0, D // 16, zlane, c)

    lax.fori_loop(0, WR, zrow, 0)

    def zcopy(t, c):
        cidx = t * NS + sid

        @pl.when(cidx < NCHN)
        def _():
            pltpu.sync_copy(zb, shared.at[pl.ds(cidx * WR, WR)])
        return c

    lax.fori_loop(0, NTRIP, zcopy, 0)


def _writeout(shared, ob, out_hbm, cid, sid):
    def wout(t, c):
        cidx = t * NS + sid

        @pl.when(cidx < NCHN)
        def _():
            r0 = cidx * WR
            pltpu.sync_copy(shared.at[pl.ds(r0, WR)], ob)
            pltpu.sync_copy(ob, out_hbm.at[cid, pl.ds(r0, WR)])
        return c

    lax.fori_loop(0, NTRIP, wout, 0)


def _make_scatter(ne, cb, half_off):
    epw = ne // NW
    nchunk = epw // cb

    def body(m2_hbm, row_hbm, agg_hbm, ridx0, mb0, ridx1, mb1, ob,
             shared, semL0, semL1, semS0, semS1):
        cid = lax.axis_index("c")
        sid = lax.axis_index("s")
        _zero_shared(shared, ob, sid)
        plsc.subcore_barrier()
        base = (cid * NS + sid) * epw
        ridx, mb = (ridx0, ridx1), (mb0, mb1)
        semL, semS = (semL0, semL1), (semS0, semS1)

        def issue(k, b):
            off = base + k * cb
            pltpu.async_copy(
                row_hbm.at[pl.ds(half_off + off, cb)], ridx[b], semL[b])
            pltpu.async_copy(m2_hbm.at[pl.ds(off, cb)], mb[b], semL[b])

        issue(0, 0)

        def outer(t, carry):
            for b in (0, 1):
                k = 2 * t + b
                nb = 1 - b

                @pl.when(k + 1 < nchunk)
                def _issue_next():
                    @pl.when(k >= 1)
                    def _wait_scat():
                        pltpu.make_async_copy(
                            mb[nb], shared.at[pl.ds(0, cb)], semS[nb]).wait()
                    issue(k + 1, nb)

                @pl.when(k < nchunk)
                def _drain():
                    pltpu.make_async_copy(
                        row_hbm.at[pl.ds(0, cb)], ridx[b], semL[b]).wait()
                    pltpu.make_async_copy(
                        m2_hbm.at[pl.ds(0, cb)], mb[b], semL[b]).wait()
                    pltpu.async_copy(
                        mb[b], shared.at[ridx[b]], semS[b], add=True)
            return carry

        lax.fori_loop(0, (nchunk + 1) // 2, outer, 0)
        pltpu.make_async_copy(mb[0], shared.at[pl.ds(0, cb)], semS[0]).wait()
        pltpu.make_async_copy(mb[1], shared.at[pl.ds(0, cb)], semS[1]).wait()
        plsc.subcore_barrier()
        _writeout(shared, ob, agg_hbm, cid, sid)

    @jax.jit
    def run(m2, row):
        return pl.kernel(
            body,
            out_type=_sds((NC, NN, D)),
            mesh=_MESH,
            compiler_params=_SC_PARAMS,
            scratch_types=[
                pltpu.VMEM((cb,), jnp.int32),
                pltpu.VMEM((cb, D), F32),
                pltpu.VMEM((cb,), jnp.int32),
                pltpu.VMEM((cb, D), F32),
                pltpu.VMEM((WR, D), F32),
                pltpu.VMEM_SHARED((NN, D), F32),
                pltpu.SemaphoreType.DMA,
                pltpu.SemaphoreType.DMA,
                pltpu.SemaphoreType.DMA,
                pltpu.SemaphoreType.DMA,
            ],
        )(m2, row)

    return run


_scatter_h0 = _make_scatter(NE2, CB2, 0)
_scatter_h1 = _make_scatter(NE2, CB2, NE2)


def _scatter_tr_body(tr0_hbm, tr1_hbm, row_hbm, xacc_hbm,
                     ridx0, trb0, ridx1, trb1, px, py, pz, pc,
                     semL0, semL1):
    cid = lax.axis_index("c")
    sid = lax.axis_index("s")
    wid = cid * NS + sid
    pacc = (px, py, pz, pc)
    for p in pacc:
        def zp(j, c, p=p):
            p[pl.ds(j * 16, 16)] = jnp.zeros((16,), F32)
            return c
        lax.fori_loop(0, NN // 16, zp, 0)
    base = sid * EPW
    ridx, trb = (ridx0, ridx1), (trb0, trb1)
    semL = (semL0, semL1)
    lane16 = lax.iota(jnp.int32, 16)

    def issue(k, b):
        off = base + k * CB
        pltpu.async_copy(
            row_hbm.at[pl.ds(cid * NE2 + off, CB)], ridx[b], semL[b])

        @pl.when(cid == 0)
        def _h0():
            pltpu.async_copy(tr0_hbm.at[pl.ds(off, CB)], trb[b], semL[b])

        @pl.when(cid == 1)
        def _h1():
            pltpu.async_copy(tr1_hbm.at[pl.ds(off, CB)], trb[b], semL[b])

    issue(0, 0)

    def outer(t, carry):
        for b in (0, 1):
            k = 2 * t + b
            nb = 1 - b

            @pl.when(k + 1 < NCHUNK)
            def _issue_next():
                issue(k + 1, nb)

            @pl.when(k < NCHUNK)
            def _drain():
                pltpu.make_async_copy(
                    row_hbm.at[pl.ds(0, CB)], ridx[b], semL[b]).wait()
                pltpu.make_async_copy(
                    tr0_hbm.at[pl.ds(0, CB)], trb[b], semL[b]).wait()

                def grp(j, c2):
                    idxv = ridx[b][pl.ds(j * 16, 16)]
                    erow = j * 16 + lane16
                    for comp, p in enumerate(pacc):
                        v = plsc.load_gather(
                            trb[b], [erow, jnp.full((16,), comp, jnp.int32)])
                        plsc.addupdate_scatter(p, [idxv], v)
                    return c2

                lax.fori_loop(0, NG, grp, 0)
        return carry

    lax.fori_loop(0, (NCHUNK + 1) // 2, outer, 0)
    for comp, p in enumerate(pacc):
        pltpu.sync_copy(p, xacc_hbm.at[comp, wid])


@jax.jit
def _scatter_tr(tr0, tr1, row):
    return pl.kernel(
        _scatter_tr_body,
        out_type=_sds((4, NW, NN)),
        mesh=_MESH,
        compiler_params=_SC_PARAMS,
        scratch_types=[
            pltpu.VMEM((CB,), jnp.int32),
            pltpu.VMEM((CB, 16), F32),
            pltpu.VMEM((CB,), jnp.int32),
            pltpu.VMEM((CB, 16), F32),
            pltpu.VMEM((NN,), F32),
            pltpu.VMEM((NN,), F32),
            pltpu.VMEM((NN,), F32),
            pltpu.VMEM((NN,), F32),
            pltpu.SemaphoreType.DMA,
            pltpu.SemaphoreType.DMA,
        ],
    )(tr0, tr1, row)


BN = 1000
BE = 2000


def _full(shape):
    return pl.BlockSpec(shape, lambda i: (0,) * len(shape))


def _pre_body(h_ref, w1at, w1bt, b1, ha_ref, hb_ref):
    h = h_ref[...]
    ha_ref[...] = jnp.dot(h, w1at[...], preferred_element_type=F32) + b1[...]
    hb_ref[...] = jnp.dot(h, w1bt[...], preferred_element_type=F32)


@jax.jit
def _pre(h, w1at, w1bt, b1):
    return pl.pallas_call(
        _pre_body,
        grid=(NN // BN,),
        in_specs=[
            pl.BlockSpec((BN, D), lambda i: (i, 0)),
            _full((D, D)), _full((D, D)), _full((1, D)),
        ],
        out_specs=[pl.BlockSpec((BN, D), lambda i: (i, 0))] * 2,
        out_shape=[_sds((NN, D))] * 2,
    )(h, w1at, w1bt, b1)


def _silu(v):
    return v * (0.5 * jnp.tanh(0.5 * v) + 0.5)


def _radial_of(cd, shape):
    lane = lax.broadcasted_iota(jnp.int32, shape, 1)
    return jnp.sum(jnp.where(lane < 3, cd * cd, 0.0), axis=1, keepdims=True)


def _edge_core(g_ref, cd_ref, wrt, w1et, w2t, b2):
    g = g_ref[...]
    cd = cd_ref[...]
    radial = _radial_of(cd, (BE, 16))
    pre = g + radial * wrt[...] + jnp.dot(
        cd, w1et[...], preferred_element_type=F32)
    m = _silu(pre.astype(jnp.bfloat16))
    return _silu(jnp.dot(m, w2t[...], preferred_element_type=F32)
                 + b2[...])


def _edge_body(g_ref, cd_ref, wrt, w1et, w2t, b2, m2_ref):
    m2_ref[...] = _edge_core(g_ref, cd_ref, wrt, w1et, w2t, b2)


@jax.jit
def _edge(g, cd, wrt, w1et, w2t, b2):
    return pl.pallas_call(
        _edge_body,
        grid=(NE2 // BE,),
        in_specs=[
            pl.BlockSpec((BE, D), lambda i: (i, 0)),
            pl.BlockSpec((BE, 16), lambda i: (i, 0)),
            _full((1, D)), _full((16, D)), _full((D, D)), _full((1, D)),
        ],
        out_specs=pl.BlockSpec((BE, D), lambda i: (i, 0)),
        out_shape=_sds((NE2, D)),
    )(g, cd, wrt, w1et, w2t, b2)


def _edge_last_body(g_ref, cd_ref, wrt, w1et, w2t, b2,
                    cw1t, cb1, cw2, m2_ref, tr_ref):
    m2 = _edge_core(g_ref, cd_ref, wrt, w1et, w2t, b2)
    m2_ref[...] = m2
    t = _silu(jnp.dot(m2, cw1t[...], preferred_element_type=F32)
              + cb1[...])
    phi = jnp.sum(t * cw2[...], axis=1, keepdims=True)
    lane = lax.broadcasted_iota(jnp.int32, (BE, 16), 1)
    tr_ref[...] = jnp.where(lane < 3, cd_ref[...] * phi,
                            jnp.where(lane == 3, 1.0, 0.0))


@jax.jit
def _edge_last(g, cd, wrt, w1et, w2t, b2, cw1t, cb1, cw2):
    return pl.pallas_call(
        _edge_last_body,
        grid=(NE2 // BE,),
        in_specs=[
            pl.BlockSpec((BE, D), lambda i: (i, 0)),
            pl.BlockSpec((BE, 16), lambda i: (i, 0)),
            _full((1, D)), _full((16, D)), _full((D, D)), _full((1, D)),
            _full((D, D)), _full((1, D)), _full((1, D)),
        ],
        out_specs=[pl.BlockSpec((BE, D), lambda i: (i, 0)),
                   pl.BlockSpec((BE, 16), lambda i: (i, 0))],
        out_shape=[_sds((NE2, D)), _sds((NE2, 16))],
    )(g, cd, wrt, w1et, w2t, b2, cw1t, cb1, cw2)


def _node_core(h_ref, a0_ref, a1_ref, a2_ref, a3_ref,
               w1ht, w1at, nb1, w2t, nb2):
    h = h_ref[...]
    agg = a0_ref[0] + a1_ref[0] + a2_ref[0] + a3_ref[0]
    o1 = _silu(jnp.dot(h, w1ht[...], preferred_element_type=F32)
                     + jnp.dot(agg, w1at[...], preferred_element_type=F32)
                     + nb1[...])
    return h + jnp.dot(o1, w2t[...], preferred_element_type=F32) + nb2[...]


def _node_body(h_ref, a0_ref, a1_ref, a2_ref, a3_ref,
               w1ht, w1at, nb1, w2t, nb2,
               ew1at, ew1bt, eb1, hn_ref, ha_ref, hb_ref):
    hn = _node_core(h_ref, a0_ref, a1_ref, a2_ref, a3_ref,
                    w1ht, w1at, nb1, w2t, nb2)
    hn_ref[...] = hn
    ha_ref[...] = jnp.dot(hn, ew1at[...], preferred_element_type=F32) + eb1[...]
    hb_ref[...] = jnp.dot(hn, ew1bt[...], preferred_element_type=F32)


_AGG_SPECS = [
    pl.BlockSpec((1, BN, D), lambda i: (0, i, 0)),
    pl.BlockSpec((1, BN, D), lambda i: (1, i, 0)),
    pl.BlockSpec((1, BN, D), lambda i: (0, i, 0)),
    pl.BlockSpec((1, BN, D), lambda i: (1, i, 0)),
]


@jax.jit
def _node(h, aggp0, aggp1, w1ht, w1at, nb1, w2t, nb2, ew1at, ew1bt, eb1):
    return pl.pallas_call(
        _node_body,
        grid=(NN // BN,),
        in_specs=[
            pl.BlockSpec((BN, D), lambda i: (i, 0)),
            *_AGG_SPECS,
            _full((D, D)), _full((D, D)), _full((1, D)),
            _full((D, D)), _full((1, D)),
            _full((D, D)), _full((D, D)), _full((1, D)),
        ],
        out_specs=[pl.BlockSpec((BN, D), lambda i: (i, 0))] * 3,
        out_shape=[_sds((NN, D))] * 3,
    )(h, aggp0, aggp0, aggp1, aggp1,
      w1ht, w1at, nb1, w2t, nb2, ew1at, ew1bt, eb1)


def _node_last_body(h_ref, a0_ref, a1_ref, a2_ref, a3_ref,
                    w1ht, w1at, nb1, w2t, nb2,
                    xt_ref, xap_ref, hn_ref, xn_ref):
    hn_ref[...] = _node_core(h_ref, a0_ref, a1_ref, a2_ref, a3_ref,
                             w1ht, w1at, nb1, w2t, nb2)
    xab = xap_ref[...]
    parts = [jnp.sum(xab[c * NW:(c + 1) * NW], axis=0, keepdims=True)
             for c in range(4)]
    cnt = jnp.clip(parts[3], 1.0, None)
    sxyz = jnp.concatenate(
        parts[:3] + [jnp.zeros((5, NN), F32)], axis=0)
    row = lax.broadcasted_iota(jnp.int32, (8, NN), 0)
    xn_ref[...] = xt_ref[...] + jnp.where(row < 3, sxyz / cnt, 0.0)


@jax.jit
def _node_last(h, aggp0, aggp1, w1ht, w1at, nb1, w2t, nb2, xt, xaccp):
    return pl.pallas_call(
        _node_last_body,
        grid=(NN // BN,),
        in_specs=[
            pl.BlockSpec((BN, D), lambda i: (i, 0)),
            *_AGG_SPECS,
            _full((D, D)), _full((D, D)), _full((1, D)),
            _full((D, D)), _full((1, D)),
            _full((8, NN)),
            _full((4 * NW, NN)),
        ],
        out_specs=[pl.BlockSpec((BN, D), lambda i: (i, 0)),
                   _full((8, NN))],
        out_shape=[_sds((NN, D)), _sds((8, NN))],
    )(h, aggp0, aggp0, aggp1, aggp1,
      w1ht, w1at, nb1, w2t, nb2, xt, xaccp)


def kernel(h, x, edges, edge_attr, params):
    row, col = edges[0], edges[1]
    xt = jnp.pad(x.T, ((0, 5), (0, 0)))
    cdp = _prep(x[:, 0], x[:, 1], x[:, 2], row, col, edge_attr)

    def esplit(p):
        w1 = p['e_w1']
        w1e16 = jnp.zeros((16, D), F32).at[3:7].set(w1[:, 2 * D + 1:].T)
        return (w1[:, :D].T, w1[:, D:2 * D].T, w1[:, 2 * D:2 * D + 1].T,
                w1e16, p['e_b1'].reshape(1, D))

    w1at, w1bt, wrt, w1et, eb1 = esplit(params[0])
    ha, hb = _pre(h, w1at, w1bt, eb1)

    cds = (cdp[0], cdp[1])
    gathers = (_gather_h0, _gather_h1)
    scatters = (_scatter_h0, _scatter_h1)

    hn = h
    for i, p in enumerate(params):
        w2t = p['e_w2'].T.astype(jnp.bfloat16)
        eb2 = p['e_b2'].reshape(1, D)
        nw1 = p['n_w1']
        nw1ht, nw1at = nw1[:, :D].T, nw1[:, D:].T
        nb1 = p['n_b1'].reshape(1, D)
        nw2t = p['n_w2'].T
        nb2 = p['n_b2'].reshape(1, D)
        gs = [gathers[hh](ha, hb, row, col) for hh in range(2)]
        if i < len(params) - 1:
            m2s = [_edge(gs[hh], cds[hh], wrt, w1et, w2t, eb2)
                   for hh in range(2)]
            aggps = [scatters[hh](m2s[hh], row) for hh in range(2)]
            w1at, w1bt, wrt, w1et, eb1 = esplit(params[i + 1])
            hn, ha, hb = _node(hn, aggps[0], aggps[1], nw1ht, nw1at, nb1,
                               nw2t, nb2, w1at, w1bt, eb1)
        else:
            cw1t = p['c_w1'].T
            cb1 = p['c_b1'].reshape(1, D)
            cw2 = p['c_w2'].reshape(1, D)
            eouts = [_edge_last(gs[hh], cds[hh], wrt, w1et, w2t, eb2,
                                cw1t, cb1, cw2) for hh in range(2)]
            aggps = [scatters[hh](eouts[hh][0], row) for hh in range(2)]
            xaccp = _scatter_tr(eouts[0][1], eouts[1][1], row)
            hn, xnt = _node_last(hn, aggps[0], aggps[1], nw1ht, nw1at, nb1,
                                 nw2t, nb2, xt, xaccp.reshape(4 * NW, NN))
    return hn, xnt[:3].T
